# Initial kernel scaffold; baseline (speedup 1.0000x reference)
import jax
import jax.numpy as jnp
from jax.experimental import pallas as pl


def kernel(pairs, adj_entity_np, adj_relation_np, entity_emb, relation_emb, W, b):
    raise NotImplementedError("write your pallas kernel here")


# speedup vs baseline: 5.5471x; 5.5471x over previous
import functools

import jax
import jax.numpy as jnp
from jax import lax
from jax.experimental import pallas as pl
from jax.experimental.pallas import tpu as pltpu
from jax.experimental.pallas import tpu_sc as plsc

B = 4096
N_ENTITY = 100000
N_RELATION = 64
DIM = 32
K = 16

NC = 2
NS = 16
NW = NC * NS
NB = B // NW

CB = 4
H1C = CB * K
H2C = CB * K * K
NCHUNK = NB // CB


def _sc_gather_body(users_hbm, items_hbm, adj_e_hbm, adj_r_hbm, emb_hbm,
                    u_out, e0_out, e1_out, e2_out, rel0_out, rel1_out,
                    us_v, it_v, urow_v, e0row_v, adj1_v, rel0_v, ent1f_v,
                    e1_v, adj2c_v, rel1c_v, ent2f_v, e2c_v, sem):
    wid = lax.axis_index("s") * NC + lax.axis_index("c")
    base = wid * NB

    pltpu.sync_copy(users_hbm.at[pl.ds(base, NB)], us_v)
    pltpu.sync_copy(items_hbm.at[pl.ds(base, NB)], it_v)

    pltpu.async_copy(emb_hbm.at[us_v], urow_v, sem).wait()
    pltpu.async_copy(emb_hbm.at[it_v], e0row_v, sem).wait()
    pltpu.sync_copy(urow_v, u_out.at[pl.ds(base, NB)])
    pltpu.sync_copy(e0row_v, e0_out.at[pl.ds(base, NB)])

    pltpu.async_copy(adj_e_hbm.at[it_v], adj1_v, sem).wait()
    pltpu.async_copy(adj_r_hbm.at[it_v], rel0_v, sem).wait()
    pltpu.sync_copy(rel0_v, rel0_out.at[pl.ds(base, NB)])

    def fl1(i, _):
        ent1f_v[pl.ds(i * K, K)] = adj1_v[i, :]
        return 0
    lax.fori_loop(0, NB, fl1, 0, unroll=4)

    def g1(c, _):
        sl = pl.ds(c * 128, 128)
        pltpu.async_copy(emb_hbm.at[ent1f_v.at[sl]], e1_v.at[sl], sem).wait()
        return 0
    lax.fori_loop(0, (NB * K) // 128, g1, 0)
    pltpu.sync_copy(e1_v, e1_out.at[pl.ds(base * K, NB * K)])

    def g2(c, _):
        h1b = c * H1C
        idx1 = ent1f_v.at[pl.ds(h1b, H1C)]
        pltpu.async_copy(adj_e_hbm.at[idx1], adj2c_v, sem).wait()
        pltpu.async_copy(adj_r_hbm.at[idx1], rel1c_v, sem).wait()
        pltpu.sync_copy(rel1c_v, rel1_out.at[pl.ds(base * K + h1b, H1C)])

        def fl2(i, _):
            ent2f_v[pl.ds(i * K, K)] = adj2c_v[i, :]
            return 0
        lax.fori_loop(0, H1C, fl2, 0, unroll=4)

        def g2e(s, _):
            sl = pl.ds(s * 128, 128)
            pltpu.async_copy(emb_hbm.at[ent2f_v.at[sl]], e2c_v.at[sl],
                             sem).wait()
            return 0
        lax.fori_loop(0, H2C // 128, g2e, 0)
        pltpu.sync_copy(e2c_v, e2_out.at[pl.ds(base * K * K + c * H2C, H2C)])
        return 0
    lax.fori_loop(0, NCHUNK, g2, 0)


@jax.jit
def _sc_gather(users, items, adj_e, adj_r, emb):
    mesh = plsc.VectorSubcoreMesh(core_axis_name="c", subcore_axis_name="s",
                                  num_cores=NC, num_subcores=NS)
    f32 = jnp.float32
    i32 = jnp.int32
    out_type = (
        jax.ShapeDtypeStruct((B, DIM), f32),
        jax.ShapeDtypeStruct((B, DIM), f32),
        jax.ShapeDtypeStruct((B * K, DIM), f32),
        jax.ShapeDtypeStruct((B * K * K, DIM), f32),
        jax.ShapeDtypeStruct((B, K), i32),
        jax.ShapeDtypeStruct((B * K, K), i32),
    )
    scratch = [
        pltpu.VMEM((NB,), i32),
        pltpu.VMEM((NB,), i32),
        pltpu.VMEM((NB, DIM), f32),
        pltpu.VMEM((NB, DIM), f32),
        pltpu.VMEM((NB, K), i32),
        pltpu.VMEM((NB, K), i32),
        pltpu.VMEM((NB * K,), i32),
        pltpu.VMEM((NB * K, DIM), f32),
        pltpu.VMEM((H1C, K), i32),
        pltpu.VMEM((H1C, K), i32),
        pltpu.VMEM((H2C,), i32),
        pltpu.VMEM((H2C, DIM), f32),
        pltpu.SemaphoreType.DMA,
    ]
    fn = pl.kernel(_sc_gather_body, out_type=out_type, mesh=mesh,
                   scratch_types=scratch,
                   compiler_params=pltpu.CompilerParams(
                       use_tc_tiling_on_sc=False))
    return fn(users, items, adj_e, adj_r, emb)


BB = 64
G1 = BB * K


def _tc_body(u_ref, e0_ref, e1_ref, e2_ref, rel0_ref, rel1_ref,
             relemb_ref, w_ref, b_ref, out_ref):
    U = u_ref[...]
    Wt = w_ref[...].T
    bb = b_ref[...]
    st = jnp.dot(U, relemb_ref[...].T,
                 preferred_element_type=jnp.float32)

    iota_r1 = lax.broadcasted_iota(jnp.int32, (G1, K, N_RELATION), 2)
    oh1 = (rel1_ref[...][:, :, None] == iota_r1).astype(jnp.float32)
    st_g = jnp.reshape(
        jnp.broadcast_to(st[:, None, :], (BB, K, N_RELATION)),
        (G1, N_RELATION))
    s1 = jnp.sum(oh1 * st_g[:, None, :], axis=-1)
    m1 = jnp.max(s1, axis=-1, keepdims=True)
    ex1 = jnp.exp(s1 - m1)
    w1 = ex1 / jnp.sum(ex1, axis=-1, keepdims=True)
    e2g = jnp.reshape(e2_ref[...], (G1, K, DIM))
    n1 = jnp.sum(w1[:, :, None] * e2g, axis=1)
    e1 = e1_ref[...]
    h1 = jax.nn.relu(jnp.dot(e1 + n1, Wt,
                             preferred_element_type=jnp.float32) + bb)

    iota_r0 = lax.broadcasted_iota(jnp.int32, (BB, K, N_RELATION), 2)
    oh0 = (rel0_ref[...][:, :, None] == iota_r0).astype(jnp.float32)
    s0 = jnp.sum(oh0 * st[:, None, :], axis=-1)
    m0 = jnp.max(s0, axis=-1, keepdims=True)
    ex0 = jnp.exp(s0 - m0)
    w0 = ex0 / jnp.sum(ex0, axis=-1, keepdims=True)
    e1g = jnp.reshape(e1, (BB, K, DIM))
    n0 = jnp.sum(w0[:, :, None] * e1g, axis=1)
    h0 = jax.nn.relu(jnp.dot(e0_ref[...] + n0, Wt,
                             preferred_element_type=jnp.float32) + bb)

    h1g = jnp.reshape(h1, (BB, K, DIM))
    n0p = jnp.sum(w0[:, :, None] * h1g, axis=1)
    outv = jnp.tanh(jnp.dot(h0 + n0p, Wt,
                            preferred_element_type=jnp.float32) + bb)
    pred = jax.nn.sigmoid(jnp.sum(U * outv, axis=-1, keepdims=True))
    out_ref[...] = pred


@jax.jit
def _tc_compute(u, e0, e1, e2, rel0, rel1, relemb, W, b2):
    grid = (B // BB,)
    f32 = jnp.float32
    return pl.pallas_call(
        _tc_body,
        grid=grid,
        in_specs=[
            pl.BlockSpec((BB, DIM), lambda i: (i, 0)),
            pl.BlockSpec((BB, DIM), lambda i: (i, 0)),
            pl.BlockSpec((G1, DIM), lambda i: (i, 0)),
            pl.BlockSpec((G1 * K, DIM), lambda i: (i, 0)),
            pl.BlockSpec((BB, K), lambda i: (i, 0)),
            pl.BlockSpec((G1, K), lambda i: (i, 0)),
            pl.BlockSpec((N_RELATION, DIM), lambda i: (0, 0)),
            pl.BlockSpec((DIM, DIM), lambda i: (0, 0)),
            pl.BlockSpec((1, DIM), lambda i: (0, 0)),
        ],
        out_specs=pl.BlockSpec((BB, 1), lambda i: (i, 0)),
        out_shape=jax.ShapeDtypeStruct((B, 1), f32),
    )(u, e0, e1, e2, rel0, rel1, relemb, W, b2)


def kernel(pairs, adj_entity_np, adj_relation_np, entity_emb, relation_emb,
           W, b):
    users = pairs[:, 0]
    items = pairs[:, 1]
    u, e0, e1, e2, rel0, rel1 = _sc_gather(
        users, items, adj_entity_np, adj_relation_np, entity_emb)
    pred = _tc_compute(u, e0, e1, e2, rel0, rel1, relation_emb, W,
                       b.reshape(1, DIM))
    return pred.reshape(B)


# speedup vs baseline: 15.3181x; 2.7615x over previous
import numpy as np

import jax
import jax.numpy as jnp
from jax import lax
from jax.experimental import pallas as pl
from jax.experimental.pallas import tpu as pltpu
from jax.experimental.pallas import tpu_sc as plsc

B = 4096
N_ENTITY = 100000
N_RELATION = 64
DIM = 32
K = 16

NC = 2
NS = 16
NW = NC * NS
NB = B // NW

CB = 4
H1C = CB * K
H2C = CB * K * K
NCHUNK = NB // CB


def _sc_gather_body(users_hbm, items_hbm, adj_e_hbm, adj_r_hbm, emb_hbm,
                    u_out, e0_out, e1_out, e2_out, rel0_out, rel1_out,
                    us_v, it_v, urow_v, e0row_v, adj1_v, rel0_v,
                    ent1f_v, adjc_v, rel1c_v, ent2f_v, e2c_v,
                    sem_g, sem_a, sem_wr, sem_we, sem_w0):
    wid = lax.axis_index("s") * NC + lax.axis_index("c")
    base = wid * NB

    pltpu.sync_copy(users_hbm.at[pl.ds(base, NB)], us_v)
    pltpu.sync_copy(items_hbm.at[pl.ds(base, NB)], it_v)

    h_u = pltpu.async_copy(emb_hbm.at[us_v], urow_v, sem_g)
    h_e0 = pltpu.async_copy(emb_hbm.at[it_v], e0row_v, sem_g)
    h_a1 = pltpu.async_copy(adj_e_hbm.at[it_v], adj1_v, sem_a)
    h_r0 = pltpu.async_copy(adj_r_hbm.at[it_v], rel0_v, sem_a)
    h_u.wait()
    w_u = pltpu.async_copy(urow_v, u_out.at[pl.ds(base, NB)], sem_w0)
    h_e0.wait()
    w_e0 = pltpu.async_copy(e0row_v, e0_out.at[pl.ds(base, NB)], sem_w0)
    h_r0.wait()
    w_r0 = pltpu.async_copy(rel0_v, rel0_out.at[pl.ds(base, NB)], sem_w0)
    h_a1.wait()

    def fl1(i, _):
        ent1f_v[pl.ds(i * K, K)] = adj1_v[i, :]
        return 0
    lax.fori_loop(0, NB, fl1, 0, unroll=4)

    e1w = []
    for h in range(2):
        hs = [pltpu.async_copy(
            emb_hbm.at[ent1f_v.at[pl.ds(h * H2C + s * 128, 128)]],
            e2c_v.at[h, pl.ds(s * 128, 128)], sem_g) for s in range(8)]
        for hh in hs:
            hh.wait()
        e1w.append(pltpu.async_copy(
            e2c_v.at[h], e1_out.at[pl.ds(base * K + h * H2C, H2C)], sem_we))

    adj_h = {}
    g_h = {}
    wr_h = {}
    we_h = {}

    def fire_adj(c):
        p = c % 2
        idx = ent1f_v.at[pl.ds(c * H1C, H1C)]
        adj_h[c] = (
            pltpu.async_copy(adj_e_hbm.at[idx], adjc_v.at[p], sem_a),
            pltpu.async_copy(adj_r_hbm.at[idx], rel1c_v.at[p], sem_a),
        )

    fire_adj(0)
    for c in range(NCHUNK):
        p = c % 2
        adj_h[c][0].wait()
        adj_h[c][1].wait()
        wr_h[c] = pltpu.async_copy(
            rel1c_v.at[p], rel1_out.at[pl.ds(base * K + c * H1C, H1C)],
            sem_wr)

        def fl2(i, _, p=p):
            ent2f_v[p, pl.ds(i * K, K)] = adjc_v[p, i, :]
            return 0
        lax.fori_loop(0, H1C, fl2, 0, unroll=4)

        if c < 2:
            e1w[c].wait()
        else:
            we_h[c - 2].wait()
        g_h[c] = [pltpu.async_copy(
            emb_hbm.at[ent2f_v.at[p, pl.ds(s * 128, 128)]],
            e2c_v.at[p, pl.ds(s * 128, 128)], sem_g) for s in range(8)]
        if c + 1 < NCHUNK:
            if c >= 1:
                wr_h[c - 1].wait()
            fire_adj(c + 1)
        if c >= 1:
            for hh in g_h[c - 1]:
                hh.wait()
            we_h[c - 1] = pltpu.async_copy(
                e2c_v.at[1 - p],
                e2_out.at[pl.ds(base * K * K + (c - 1) * H2C, H2C)], sem_we)

    c_last = NCHUNK - 1
    for hh in g_h[c_last]:
        hh.wait()
    we_h[c_last] = pltpu.async_copy(
        e2c_v.at[c_last % 2],
        e2_out.at[pl.ds(base * K * K + c_last * H2C, H2C)], sem_we)
    we_h[c_last - 1].wait()
    we_h[c_last].wait()
    wr_h[c_last - 1].wait()
    wr_h[c_last].wait()
    w_u.wait()
    w_e0.wait()
    w_r0.wait()


@jax.jit
def _sc_gather(users, items, adj_e, adj_r, emb):
    mesh = plsc.VectorSubcoreMesh(core_axis_name="c", subcore_axis_name="s",
                                  num_cores=NC, num_subcores=NS)
    f32 = jnp.float32
    i32 = jnp.int32
    out_type = (
        jax.ShapeDtypeStruct((B, DIM), f32),
        jax.ShapeDtypeStruct((B, DIM), f32),
        jax.ShapeDtypeStruct((B * K, DIM), f32),
        jax.ShapeDtypeStruct((B * K * K, DIM), f32),
        jax.ShapeDtypeStruct((B, K), i32),
        jax.ShapeDtypeStruct((B * K, K), i32),
    )
    scratch = [
        pltpu.VMEM((NB,), i32),
        pltpu.VMEM((NB,), i32),
        pltpu.VMEM((NB, DIM), f32),
        pltpu.VMEM((NB, DIM), f32),
        pltpu.VMEM((NB, K), i32),
        pltpu.VMEM((NB, K), i32),
        pltpu.VMEM((NB * K,), i32),
        pltpu.VMEM((2, H1C, K), i32),
        pltpu.VMEM((2, H1C, K), i32),
        pltpu.VMEM((2, H2C), i32),
        pltpu.VMEM((2, H2C, DIM), f32),
        pltpu.SemaphoreType.DMA,
        pltpu.SemaphoreType.DMA,
        pltpu.SemaphoreType.DMA,
        pltpu.SemaphoreType.DMA,
        pltpu.SemaphoreType.DMA,
    ]
    fn = pl.kernel(_sc_gather_body, out_type=out_type, mesh=mesh,
                   scratch_types=scratch,
                   compiler_params=pltpu.CompilerParams(
                       use_tc_tiling_on_sc=False))
    return fn(users, items, adj_e, adj_r, emb)


BB = 128
G1 = BB * K

_M2 = np.equal(np.arange(BB * K)[:, None] // K,
               np.arange(BB)[None, :]).astype(np.float32)
_REP = np.kron(np.eye(K), np.ones((1, DIM))).astype(np.float32)
_COL = np.kron(np.ones((K, 1)), np.eye(DIM)).astype(np.float32)
_P = np.tile(np.eye(K), (1, BB)).astype(np.float32)
_M = np.kron(np.eye(BB), np.ones((1, K))).astype(np.float32)


def _tc_body(u_ref, e0_ref, e1_ref, e2_ref, rel0_ref, rel1_ref,
             relembt_ref, wt_ref, b_ref, m2_ref, rep_ref, col_ref,
             pm_ref, mm_ref, out_ref):
    U = u_ref[...]
    Wt = wt_ref[...]
    bb = b_ref[...]
    m2 = m2_ref[...]
    rep = rep_ref[...]
    col = col_ref[...]
    pm = pm_ref[...]
    mm = mm_ref[...]

    st = jnp.dot(U, relembt_ref[...],
                 preferred_element_type=jnp.float32)
    mx = jnp.max(st, axis=1, keepdims=True)
    exst = jnp.exp(st - mx)

    exst_g = jnp.dot(m2, exst, preferred_element_type=jnp.float32)
    ex1 = jnp.take_along_axis(exst_g, rel1_ref[...], axis=1)
    w1 = ex1 / jnp.sum(ex1, axis=1, keepdims=True)
    w1e = jnp.dot(w1, rep, preferred_element_type=jnp.float32)
    n1 = jnp.dot(e2_ref[...] * w1e, col,
                 preferred_element_type=jnp.float32)
    e1 = e1_ref[...]
    h1 = jax.nn.relu(jnp.dot(e1 + n1, Wt,
                             preferred_element_type=jnp.float32) + bb)

    ex0 = jnp.take_along_axis(exst, rel0_ref[...], axis=1)
    w0 = ex0 / jnp.sum(ex0, axis=1, keepdims=True)
    w0sel = jnp.dot(w0, pm, preferred_element_type=jnp.float32) * mm
    n0 = jnp.dot(w0sel, e1, preferred_element_type=jnp.float32)
    h0 = jax.nn.relu(jnp.dot(e0_ref[...] + n0, Wt,
                             preferred_element_type=jnp.float32) + bb)

    n0p = jnp.dot(w0sel, h1, preferred_element_type=jnp.float32)
    outv = jnp.tanh(jnp.dot(h0 + n0p, Wt,
                            preferred_element_type=jnp.float32) + bb)
    pred = jax.nn.sigmoid(jnp.sum(U * outv, axis=-1, keepdims=True))
    out_ref[...] = pred


def _tc_specs():
    return [
        pl.BlockSpec((BB, DIM), lambda i: (i, 0)),
        pl.BlockSpec((BB, DIM), lambda i: (i, 0)),
        pl.BlockSpec((G1, DIM), lambda i: (i, 0)),
        pl.BlockSpec((G1, K * DIM), lambda i: (i, 0)),
        pl.BlockSpec((BB, K), lambda i: (i, 0)),
        pl.BlockSpec((G1, K), lambda i: (i, 0)),
        pl.BlockSpec((DIM, N_RELATION), lambda i: (0, 0)),
        pl.BlockSpec((DIM, DIM), lambda i: (0, 0)),
        pl.BlockSpec((1, DIM), lambda i: (0, 0)),
        pl.BlockSpec((G1, BB), lambda i: (0, 0)),
        pl.BlockSpec((K, K * DIM), lambda i: (0, 0)),
        pl.BlockSpec((K * DIM, DIM), lambda i: (0, 0)),
        pl.BlockSpec((K, G1), lambda i: (0, 0)),
        pl.BlockSpec((BB, G1), lambda i: (0, 0)),
    ]


@jax.jit
def _tc_compute(u, e0, e1, e2l, rel0, rel1, relembt, Wt, b2):
    return pl.pallas_call(
        _tc_body,
        grid=(B // BB,),
        in_specs=_tc_specs(),
        out_specs=pl.BlockSpec((BB, 1), lambda i: (i, 0)),
        out_shape=jax.ShapeDtypeStruct((B, 1), jnp.float32),
    )(u, e0, e1, e2l, rel0, rel1, relembt, Wt, b2,
      jnp.asarray(_M2), jnp.asarray(_REP), jnp.asarray(_COL),
      jnp.asarray(_P), jnp.asarray(_M))


def kernel(pairs, adj_entity_np, adj_relation_np, entity_emb, relation_emb,
           W, b):
    users = pairs[:, 0]
    items = pairs[:, 1]
    u, e0, e1, e2, rel0, rel1 = _sc_gather(
        users, items, adj_entity_np, adj_relation_np, entity_emb)
    e2l = e2.reshape(B * K, K * DIM)
    pred = _tc_compute(u, e0, e1, e2l, rel0, rel1, relation_emb.T, W.T,
                       b.reshape(1, DIM))
    return pred.reshape(B)
